# Initial kernel scaffold; baseline (speedup 1.0000x reference)
import jax
import jax.numpy as jnp
from jax.experimental import pallas as pl


def kernel(x, qkv_w, qkv_dw_w, proj_w, temperature, attn_mix, ln_w, ln_b, ff_in_w, g1_pw, g1_dw, g2_pw, g2_dw, g3_pw, g3_dw):
    raise NotImplementedError("write your pallas kernel here")


# speedup vs baseline: 1.2543x; 1.2543x over previous
import jax
import jax.numpy as jnp
from jax.experimental import pallas as pl
from jax.experimental.pallas import tpu as pltpu

C = 96
HEADS = 4
CH = 24
HH = 384
WW = 384
TH = 8
G = HH // TH
CC = CH * CH
KKS = (CC * 1 // 2, CC * 2 // 3, CC * 3 // 4, CC * 4 // 5)


def _p1_body(xp_ref, xc_ref, xn_ref, wq_ref, wd_ref, v_ref, gram_ref,
             sqq_ref, sqk_ref):
    i = pl.program_id(0)
    xw = jnp.concatenate([xp_ref[:, TH - 1:, :], xc_ref[...], xn_ref[:, :1, :]],
                         axis=1)
    g = i * TH - 1 + jax.lax.broadcasted_iota(jnp.int32, (1, TH + 2, 1), 1)
    xw = jnp.where((g >= 0) & (g < HH), xw, 0.0)
    xwp = jnp.pad(xw, ((0, 0), (0, 0), (1, 1)))
    pwp = jnp.dot(wq_ref[...], xwp.reshape(C, (TH + 2) * (WW + 2)),
                  preferred_element_type=jnp.float32).reshape(3 * C, TH + 2, WW + 2)
    wd = wd_ref[...]
    acc = jnp.zeros((3 * C, TH, WW), jnp.float32)
    for ky in range(3):
        for kx in range(3):
            acc = acc + wd[:, ky, kx][:, None, None] * pwp[:, ky:ky + TH, kx:kx + WW]
    v_ref[...] = acc[2 * C:]
    acc2 = acc.reshape(3 * C, TH * WW)
    ghs = []
    for h in range(HEADS):
        qh = acc2[CH * h:CH * (h + 1)]
        kh = acc2[C + CH * h:C + CH * (h + 1)]
        ghs.append(jax.lax.dot_general(qh, kh, (((1,), (1,)), ((), ())),
                                       preferred_element_type=jnp.float32))
    gpart = jnp.concatenate(ghs, axis=0)
    qs = jnp.pad(jnp.sum(acc2[:C] ** 2, axis=1, keepdims=True), ((0, 0), (0, 7)))
    ksr = jax.lax.dot_general(jnp.ones((1, TH * WW), jnp.float32),
                              acc2[C:2 * C] ** 2, (((1,), (1,)), ((), ())),
                              preferred_element_type=jnp.float32)
    ksr = jnp.pad(ksr, ((0, 7), (0, 128 - C)))

    @pl.when(i == 0)
    def _():
        gram_ref[...] = gpart
        sqq_ref[...] = qs
        sqk_ref[...] = ksr

    @pl.when(i > 0)
    def _():
        gram_ref[...] = gram_ref[...] + gpart
        sqq_ref[...] = sqq_ref[...] + qs
        sqk_ref[...] = sqk_ref[...] + ksr


def _p2_body(gram_ref, sqq_ref, sqk_ref, temp_ref, mix_ref, wproj_ref, m_ref):
    r24 = jax.lax.broadcasted_iota(jnp.int32, (CH, CC), 0)
    c24 = jax.lax.broadcasted_iota(jnp.int32, (CH, CC), 1)
    E1 = jnp.where(r24 == c24 // CH, 1.0, 0.0)
    E2t = jnp.where(r24 == c24 % CH, 1.0, 0.0)
    r576 = jax.lax.broadcasted_iota(jnp.int32, (CC, CH), 0)
    c576 = jax.lax.broadcasted_iota(jnp.int32, (CC, CH), 1)
    E1c = jnp.where(c576 == r576 // CH, 1.0, 0.0)
    E2c = jnp.where(c576 == r576 % CH, 1.0, 0.0)
    ii = jax.lax.broadcasted_iota(jnp.int32, (CC, CC), 0)
    jj = jax.lax.broadcasted_iota(jnp.int32, (CC, CC), 1)
    nq = jnp.maximum(jnp.sqrt(sqq_ref[:, 0:1]), 1e-12)
    nk = jnp.maximum(jnp.sqrt(sqk_ref[0:1, :C]), 1e-12)
    blocks = []
    for h in range(HEADS):
        gh = gram_ref[CH * h:CH * (h + 1), :]
        nprod = jnp.dot(nq[CH * h:CH * (h + 1), :], nk[:, CH * h:CH * (h + 1)],
                        preferred_element_type=jnp.float32)
        th = temp_ref[h:h + 1, 0:1]
        ath = gh / nprod * th
        fr = jnp.sum(E1 * jnp.dot(ath, E2t, preferred_element_type=jnp.float32),
                     axis=0, keepdims=True)
        fc = jnp.sum(jnp.dot(E1c, ath, preferred_element_type=jnp.float32) * E2c,
                     axis=1, keepdims=True)
        cmp = jnp.where((fr > fc) | ((fr == fc) & (jj < ii)), 1.0, 0.0)
        rank = jnp.sum(cmp, axis=1, keepdims=True)
        amax = jnp.max(fc)
        e = jnp.exp(fc - amax)
        coef = jnp.zeros((CC, 1), jnp.float32)
        for l, kk in enumerate(KKS):
            mask = jnp.where(rank < float(kk), 1.0, 0.0)
            denom = jnp.sum(mask * e)
            coef = coef + mix_ref[l:l + 1, 0:1] * mask / denom
        s = coef * e
        spa_h = jnp.dot(E1, s * E2c, preferred_element_type=jnp.float32)
        blocks.append(jnp.pad(spa_h, ((0, 0), (CH * h, C - CH * (h + 1)))))
    bd = jnp.concatenate(blocks, axis=0)
    m_ref[...] = jnp.dot(wproj_ref[...], bd, preferred_element_type=jnp.float32)


def _p3_body(xp_ref, xc_ref, xn_ref, vp_ref, vc_ref, vn_ref, m_ref,
             lnw_ref, lnb_ref, wff_ref, w1p_ref, w1d_ref, w2p_ref, w2d_ref,
             w3p_ref, w3d_ref, out_ref):
    i = pl.program_id(0)
    WR = TH + 8
    xw = jnp.concatenate([xp_ref[:, TH - 4:, :], xc_ref[...], xn_ref[:, :4, :]],
                         axis=1)
    vw = jnp.concatenate([vp_ref[:, TH - 4:, :], vc_ref[...], vn_ref[:, :4, :]],
                         axis=1)
    g = i * TH - 4 + jax.lax.broadcasted_iota(jnp.int32, (1, WR, 1), 1)
    valid = (g >= 0) & (g < HH)
    xw = jnp.where(valid, xw, 0.0)
    vw = jnp.where(valid, vw, 0.0)
    N = WR * WW
    xa = jnp.dot(m_ref[...], vw.reshape(C, N),
                 preferred_element_type=jnp.float32) + xw.reshape(C, N)
    mu = jnp.mean(xa, axis=0, keepdims=True)
    xm = xa - mu
    var = jnp.mean(xm * xm, axis=0, keepdims=True)
    y = lnw_ref[...] * xm / jnp.sqrt(var + 1e-6) + lnb_ref[...]
    t = jnp.dot(wff_ref[...], y, preferred_element_type=jnp.float32)
    t = 0.5 * t * (1.0 + jax.lax.erf(t * 0.7071067811865476))
    t = t.reshape(2 * C, WR, WW) * jnp.where(valid, 1.0, 0.0)
    t1 = t[:C]
    t2 = t[C:]
    t1p = jnp.pad(t1, ((0, 0), (0, 0), (1, 1)))
    a1p = jnp.dot(w1p_ref[...], t1p.reshape(C, WR * (WW + 2)),
                  preferred_element_type=jnp.float32).reshape(C, WR, WW + 2)
    w1d = w1d_ref[...]
    x1 = jnp.zeros((C, TH + 2, WW), jnp.float32)
    for ky in range(3):
        for kx in range(3):
            x1 = x1 + w1d[:, ky, kx][:, None, None] * a1p[:, 2 + ky:4 + TH + ky, kx:kx + WW]
    HW = WR // 2
    W2 = WW // 2
    pj = jax.lax.broadcasted_iota(jnp.int32, (WW, W2), 0)
    pk = jax.lax.broadcasted_iota(jnp.int32, (WW, W2), 1)
    P = jnp.where(pj // 2 == pk, 1.0, 0.0)
    tw = jnp.dot(t2.reshape(C * WR, WW), P,
                 preferred_element_type=jnp.float32).reshape(C, HW, 2, W2)
    p = (tw[:, :, 0, :] + tw[:, :, 1, :]) * 0.25
    pp = jnp.pad(p, ((0, 0), (0, 0), (1, 1)))
    a2p = jnp.dot(w2p_ref[...], pp.reshape(C, HW * (W2 + 2)),
                  preferred_element_type=jnp.float32).reshape(C, HW, W2 + 2)
    w2d = w2d_ref[...]
    HO = TH // 2 + 2
    x2h = jnp.zeros((C, HO, W2), jnp.float32)
    for ky in range(3):
        for kx in range(3):
            x2h = x2h + w2d[:, ky, kx][:, None, None] * a2p[:, ky:ky + HO, kx:kx + W2]
    x2w = jnp.dot(x2h.reshape(C * HO, W2), P.T,
                  preferred_element_type=jnp.float32).reshape(C, HO, 1, WW)
    x2f = jnp.concatenate([x2w, x2w], axis=2).reshape(C, 2 * HO, WW)[:, 1:1 + TH + 2, :]
    prodp = jnp.pad(x1 * x2f, ((0, 0), (0, 0), (1, 1)))
    a3p = jnp.dot(w3p_ref[...], prodp.reshape(C, (TH + 2) * (WW + 2)),
                  preferred_element_type=jnp.float32).reshape(C, TH + 2, WW + 2)
    w3d = w3d_ref[...]
    ff = jnp.zeros((C, TH, WW), jnp.float32)
    for ky in range(3):
        for kx in range(3):
            ff = ff + w3d[:, ky, kx][:, None, None] * a3p[:, ky:ky + TH, kx:kx + WW]
    out_ref[...] = ff + xa.reshape(C, WR, WW)[:, 4:4 + TH, :]


def kernel(x, qkv_w, qkv_dw_w, proj_w, temperature, attn_mix, ln_w, ln_b,
           ff_in_w, g1_pw, g1_dw, g2_pw, g2_dw, g3_pw, g3_dw):
    xr = x.reshape(C, HH, WW)
    w_qkv = qkv_w.reshape(3 * C, C)
    w_dw = qkv_dw_w.reshape(3 * C, 3, 3)

    row_spec_p = pl.BlockSpec((C, TH, WW), lambda i: (0, jnp.maximum(i - 1, 0), 0))
    row_spec_c = pl.BlockSpec((C, TH, WW), lambda i: (0, i, 0))
    row_spec_n = pl.BlockSpec((C, TH, WW), lambda i: (0, jnp.minimum(i + 1, G - 1), 0))

    cparams = pltpu.CompilerParams(vmem_limit_bytes=64 * 1024 * 1024)

    v, gram, sqq, sqk = pl.pallas_call(
        _p1_body,
        grid=(G,),
        compiler_params=cparams,
        in_specs=[
            row_spec_p, row_spec_c, row_spec_n,
            pl.BlockSpec((3 * C, C), lambda i: (0, 0)),
            pl.BlockSpec((3 * C, 3, 3), lambda i: (0, 0, 0)),
        ],
        out_specs=[
            pl.BlockSpec((C, TH, WW), lambda i: (0, i, 0)),
            pl.BlockSpec((C, CH), lambda i: (0, 0)),
            pl.BlockSpec((C, 8), lambda i: (0, 0)),
            pl.BlockSpec((8, 128), lambda i: (0, 0)),
        ],
        out_shape=[
            jax.ShapeDtypeStruct((C, HH, WW), jnp.float32),
            jax.ShapeDtypeStruct((C, CH), jnp.float32),
            jax.ShapeDtypeStruct((C, 8), jnp.float32),
            jax.ShapeDtypeStruct((8, 128), jnp.float32),
        ],
    )(xr, xr, xr, w_qkv, w_dw)

    m = pl.pallas_call(
        _p2_body,
        out_shape=jax.ShapeDtypeStruct((C, C), jnp.float32),
    )(gram, sqq, sqk, temperature.reshape(HEADS, 1), attn_mix.reshape(4, 1),
      proj_w.reshape(C, C))

    out = pl.pallas_call(
        _p3_body,
        grid=(G,),
        compiler_params=cparams,
        in_specs=[
            row_spec_p, row_spec_c, row_spec_n,
            row_spec_p, row_spec_c, row_spec_n,
            pl.BlockSpec((C, C), lambda i: (0, 0)),
            pl.BlockSpec((C, 1), lambda i: (0, 0)),
            pl.BlockSpec((C, 1), lambda i: (0, 0)),
            pl.BlockSpec((2 * C, C), lambda i: (0, 0)),
            pl.BlockSpec((C, C), lambda i: (0, 0)),
            pl.BlockSpec((C, 3, 3), lambda i: (0, 0, 0)),
            pl.BlockSpec((C, C), lambda i: (0, 0)),
            pl.BlockSpec((C, 3, 3), lambda i: (0, 0, 0)),
            pl.BlockSpec((C, C), lambda i: (0, 0)),
            pl.BlockSpec((C, 3, 3), lambda i: (0, 0, 0)),
        ],
        out_specs=pl.BlockSpec((C, TH, WW), lambda i: (0, i, 0)),
        out_shape=jax.ShapeDtypeStruct((C, HH, WW), jnp.float32),
    )(xr, xr, xr, v, v, v, m,
      ln_w.reshape(C, 1), ln_b.reshape(C, 1), ff_in_w.reshape(2 * C, C),
      g1_pw.reshape(C, C), g1_dw.reshape(C, 3, 3),
      g2_pw.reshape(C, C), g2_dw.reshape(C, 3, 3),
      g3_pw.reshape(C, C), g3_dw.reshape(C, 3, 3))

    return out.reshape(1, C, HH, WW)


# speedup vs baseline: 1.2550x; 1.0005x over previous
import jax
import jax.numpy as jnp
from jax.experimental import pallas as pl
from jax.experimental.pallas import tpu as pltpu

C = 96
HEADS = 4
CH = 24
HH = 384
WW = 384
TH = 8
G = HH // TH
CC = CH * CH
KKS = (CC * 1 // 2, CC * 2 // 3, CC * 3 // 4, CC * 4 // 5)


def _bdot(a, b):
    return jnp.dot(a, b, preferred_element_type=jnp.float32)


def _p1_body(xp_ref, xc_ref, xn_ref, wq_ref, wd_ref, v_ref, gram_ref,
             sqq_ref, sqk_ref):
    i = pl.program_id(0)
    xw = jnp.concatenate([xp_ref[:, TH - 1:, :], xc_ref[...], xn_ref[:, :1, :]],
                         axis=1)
    g = i * TH - 1 + jax.lax.broadcasted_iota(jnp.int32, (1, TH + 2, 1), 1)
    xw = jnp.where((g >= 0) & (g < HH), xw, 0.0)
    xwp = jnp.pad(xw, ((0, 0), (0, 0), (1, 1)))
    pwp = _bdot(wq_ref[...], xwp.reshape(C, (TH + 2) * (WW + 2))
                ).reshape(3 * C, TH + 2, WW + 2)
    wd = wd_ref[...]
    acc = jnp.zeros((3 * C, TH, WW), jnp.float32)
    for ky in range(3):
        for kx in range(3):
            acc = acc + wd[:, ky, kx][:, None, None] * pwp[:, ky:ky + TH, kx:kx + WW]
    v_ref[...] = acc[2 * C:]
    acc2 = acc.reshape(3 * C, TH * WW)
    ghs = []
    for h in range(HEADS):
        qh = acc2[CH * h:CH * (h + 1)]
        kh = acc2[C + CH * h:C + CH * (h + 1)]
        ghs.append(jax.lax.dot_general(qh, kh, (((1,), (1,)), ((), ())),
                                       preferred_element_type=jnp.float32))
    gpart = jnp.concatenate(ghs, axis=0)
    qs = jnp.pad(jnp.sum(acc2[:C] ** 2, axis=1, keepdims=True), ((0, 0), (0, 7)))
    ksr = jax.lax.dot_general(jnp.ones((1, TH * WW), jnp.float32),
                              acc2[C:2 * C] ** 2, (((1,), (1,)), ((), ())),
                              preferred_element_type=jnp.float32)
    ksr = jnp.pad(ksr, ((0, 7), (0, 128 - C)))

    @pl.when(i == 0)
    def _():
        gram_ref[...] = gpart
        sqq_ref[...] = qs
        sqk_ref[...] = ksr

    @pl.when(i > 0)
    def _():
        gram_ref[...] = gram_ref[...] + gpart
        sqq_ref[...] = sqq_ref[...] + qs
        sqk_ref[...] = sqk_ref[...] + ksr


def _p2_body(gram_ref, sqq_ref, sqk_ref, temp_ref, mix_ref, wproj_ref, m_ref):
    r24 = jax.lax.broadcasted_iota(jnp.int32, (CH, CC), 0)
    c24 = jax.lax.broadcasted_iota(jnp.int32, (CH, CC), 1)
    E1 = jnp.where(r24 == c24 // CH, 1.0, 0.0)
    E2t = jnp.where(r24 == c24 % CH, 1.0, 0.0)
    r576 = jax.lax.broadcasted_iota(jnp.int32, (CC, CH), 0)
    c576 = jax.lax.broadcasted_iota(jnp.int32, (CC, CH), 1)
    E1c = jnp.where(c576 == r576 // CH, 1.0, 0.0)
    E2c = jnp.where(c576 == r576 % CH, 1.0, 0.0)
    ii = jax.lax.broadcasted_iota(jnp.int32, (CC, CC), 0)
    jj = jax.lax.broadcasted_iota(jnp.int32, (CC, CC), 1)
    nq = jnp.maximum(jnp.sqrt(sqq_ref[:, 0:1]), 1e-12)
    nk = jnp.maximum(jnp.sqrt(sqk_ref[0:1, :C]), 1e-12)
    blocks = []
    for h in range(HEADS):
        gh = gram_ref[CH * h:CH * (h + 1), :]
        nprod = jnp.dot(nq[CH * h:CH * (h + 1), :], nk[:, CH * h:CH * (h + 1)],
                        preferred_element_type=jnp.float32)
        th = temp_ref[h:h + 1, 0:1]
        ath = gh / nprod * th
        fr = jnp.sum(E1 * jnp.dot(ath, E2t, preferred_element_type=jnp.float32),
                     axis=0, keepdims=True)
        fc = jnp.sum(jnp.dot(E1c, ath, preferred_element_type=jnp.float32) * E2c,
                     axis=1, keepdims=True)
        cmp = jnp.where((fr > fc) | ((fr == fc) & (jj < ii)), 1.0, 0.0)
        rank = jnp.sum(cmp, axis=1, keepdims=True)
        amax = jnp.max(fc)
        e = jnp.exp(fc - amax)
        coef = jnp.zeros((CC, 1), jnp.float32)
        for l, kk in enumerate(KKS):
            mask = jnp.where(rank < float(kk), 1.0, 0.0)
            denom = jnp.sum(mask * e)
            coef = coef + mix_ref[l:l + 1, 0:1] * mask / denom
        s = coef * e
        spa_h = jnp.dot(E1, s * E2c, preferred_element_type=jnp.float32)
        blocks.append(jnp.pad(spa_h, ((0, 0), (CH * h, C - CH * (h + 1)))))
    bd = jnp.concatenate(blocks, axis=0)
    m_ref[...] = jnp.dot(wproj_ref[...], bd, preferred_element_type=jnp.float32)


def _p3_body(xp_ref, xc_ref, xn_ref, vp_ref, vc_ref, vn_ref, m_ref,
             lnw_ref, lnb_ref, wff_ref, w1p_ref, w1d_ref, w2p_ref, w2d_ref,
             w3p_ref, w3d_ref, out_ref):
    i = pl.program_id(0)
    WR = TH + 8
    xw = jnp.concatenate([xp_ref[:, TH - 4:, :], xc_ref[...], xn_ref[:, :4, :]],
                         axis=1)
    vw = jnp.concatenate([vp_ref[:, TH - 4:, :], vc_ref[...], vn_ref[:, :4, :]],
                         axis=1)
    g = i * TH - 4 + jax.lax.broadcasted_iota(jnp.int32, (1, WR, 1), 1)
    valid = (g >= 0) & (g < HH)
    xw = jnp.where(valid, xw, 0.0)
    vw = jnp.where(valid, vw, 0.0)
    N = WR * WW
    xa = _bdot(m_ref[...], vw.reshape(C, N)) + xw.reshape(C, N)
    mu = jnp.mean(xa, axis=0, keepdims=True)
    xm = xa - mu
    var = jnp.mean(xm * xm, axis=0, keepdims=True)
    y = lnw_ref[...] * xm / jnp.sqrt(var + 1e-6) + lnb_ref[...]
    t = _bdot(wff_ref[...], y)
    t = 0.5 * t * (1.0 + jax.lax.erf(t * 0.7071067811865476))
    t = t.reshape(2 * C, WR, WW) * jnp.where(valid, 1.0, 0.0)
    t1 = t[:C]
    t2 = t[C:]
    t1p = jnp.pad(t1, ((0, 0), (0, 0), (1, 1)))
    a1p = _bdot(w1p_ref[...], t1p.reshape(C, WR * (WW + 2))).reshape(C, WR, WW + 2)
    w1d = w1d_ref[...]
    x1 = jnp.zeros((C, TH + 2, WW), jnp.float32)
    for ky in range(3):
        for kx in range(3):
            x1 = x1 + w1d[:, ky, kx][:, None, None] * a1p[:, 2 + ky:4 + TH + ky, kx:kx + WW]
    HW = WR // 2
    W2 = WW // 2
    pj = jax.lax.broadcasted_iota(jnp.int32, (WW, W2), 0)
    pk = jax.lax.broadcasted_iota(jnp.int32, (WW, W2), 1)
    P = jnp.where(pj // 2 == pk, 1.0, 0.0)
    tw = jnp.dot(t2.reshape(C * WR, WW), P,
                 preferred_element_type=jnp.float32).reshape(C, HW, 2, W2)
    p = (tw[:, :, 0, :] + tw[:, :, 1, :]) * 0.25
    pp = jnp.pad(p, ((0, 0), (0, 0), (1, 1)))
    a2p = _bdot(w2p_ref[...], pp.reshape(C, HW * (W2 + 2))).reshape(C, HW, W2 + 2)
    w2d = w2d_ref[...]
    HO = TH // 2 + 2
    x2h = jnp.zeros((C, HO, W2), jnp.float32)
    for ky in range(3):
        for kx in range(3):
            x2h = x2h + w2d[:, ky, kx][:, None, None] * a2p[:, ky:ky + HO, kx:kx + W2]
    x2w = jnp.dot(x2h.reshape(C * HO, W2), P.T,
                  preferred_element_type=jnp.float32).reshape(C, HO, 1, WW)
    x2f = jnp.concatenate([x2w, x2w], axis=2).reshape(C, 2 * HO, WW)[:, 1:1 + TH + 2, :]
    prodp = jnp.pad(x1 * x2f, ((0, 0), (0, 0), (1, 1)))
    a3p = _bdot(w3p_ref[...], prodp.reshape(C, (TH + 2) * (WW + 2))
                ).reshape(C, TH + 2, WW + 2)
    w3d = w3d_ref[...]
    ff = jnp.zeros((C, TH, WW), jnp.float32)
    for ky in range(3):
        for kx in range(3):
            ff = ff + w3d[:, ky, kx][:, None, None] * a3p[:, ky:ky + TH, kx:kx + WW]
    out_ref[...] = ff + xa.reshape(C, WR, WW)[:, 4:4 + TH, :]


def kernel(x, qkv_w, qkv_dw_w, proj_w, temperature, attn_mix, ln_w, ln_b,
           ff_in_w, g1_pw, g1_dw, g2_pw, g2_dw, g3_pw, g3_dw):
    xr = x.reshape(C, HH, WW)
    w_qkv = qkv_w.reshape(3 * C, C)
    w_dw = qkv_dw_w.reshape(3 * C, 3, 3)

    row_spec_p = pl.BlockSpec((C, TH, WW), lambda i: (0, jnp.maximum(i - 1, 0), 0))
    row_spec_c = pl.BlockSpec((C, TH, WW), lambda i: (0, i, 0))
    row_spec_n = pl.BlockSpec((C, TH, WW), lambda i: (0, jnp.minimum(i + 1, G - 1), 0))

    v, gram, sqq, sqk = pl.pallas_call(
        _p1_body,
        grid=(G,),
        in_specs=[
            row_spec_p, row_spec_c, row_spec_n,
            pl.BlockSpec((3 * C, C), lambda i: (0, 0)),
            pl.BlockSpec((3 * C, 3, 3), lambda i: (0, 0, 0)),
        ],
        out_specs=[
            pl.BlockSpec((C, TH, WW), lambda i: (0, i, 0)),
            pl.BlockSpec((C, CH), lambda i: (0, 0)),
            pl.BlockSpec((C, 8), lambda i: (0, 0)),
            pl.BlockSpec((8, 128), lambda i: (0, 0)),
        ],
        out_shape=[
            jax.ShapeDtypeStruct((C, HH, WW), jnp.float32),
            jax.ShapeDtypeStruct((C, CH), jnp.float32),
            jax.ShapeDtypeStruct((C, 8), jnp.float32),
            jax.ShapeDtypeStruct((8, 128), jnp.float32),
        ],
    )(xr, xr, xr, w_qkv, w_dw)

    m = pl.pallas_call(
        _p2_body,
        out_shape=jax.ShapeDtypeStruct((C, C), jnp.float32),
    )(gram, sqq, sqk, temperature.reshape(HEADS, 1), attn_mix.reshape(4, 1),
      proj_w.reshape(C, C))

    out = pl.pallas_call(
        _p3_body,
        grid=(G,),
        in_specs=[
            row_spec_p, row_spec_c, row_spec_n,
            row_spec_p, row_spec_c, row_spec_n,
            pl.BlockSpec((C, C), lambda i: (0, 0)),
            pl.BlockSpec((C, 1), lambda i: (0, 0)),
            pl.BlockSpec((C, 1), lambda i: (0, 0)),
            pl.BlockSpec((2 * C, C), lambda i: (0, 0)),
            pl.BlockSpec((C, C), lambda i: (0, 0)),
            pl.BlockSpec((C, 3, 3), lambda i: (0, 0, 0)),
            pl.BlockSpec((C, C), lambda i: (0, 0)),
            pl.BlockSpec((C, 3, 3), lambda i: (0, 0, 0)),
            pl.BlockSpec((C, C), lambda i: (0, 0)),
            pl.BlockSpec((C, 3, 3), lambda i: (0, 0, 0)),
        ],
        out_specs=pl.BlockSpec((C, TH, WW), lambda i: (0, i, 0)),
        out_shape=jax.ShapeDtypeStruct((C, HH, WW), jnp.float32),
    )(xr, xr, xr, v, v, v, m,
      ln_w.reshape(C, 1), ln_b.reshape(C, 1), ff_in_w.reshape(2 * C, C),
      g1_pw.reshape(C, C), g1_dw.reshape(C, 3, 3),
      g2_pw.reshape(C, C), g2_dw.reshape(C, 3, 3),
      g3_pw.reshape(C, C), g3_dw.reshape(C, 3, 3))

    return out.reshape(1, C, HH, WW)
